# baseline (speedup 1.0000x reference)
import dataclasses

import jax
import jax.numpy as jnp
from jax import lax
from jax.experimental import pallas as pl
from jax.experimental.pallas import tpu as pltpu
from jax.experimental.pallas import tpu_sc as plsc

N = 320000
D = 128
S = 10000
NW = 32
Q = N // NW
C = 400
NCHUNK = N // C
NV = D // 16
W = 128
HW = W // 2

NEG_INF = float("-inf")


def _body(x_hbm, b_hbm, o_hbm, xbuf0, xbuf1, idbuf0, idbuf1, bpos, staging,
          sidx, negbuf, prevbuf, sem0, sem1, stsem):
    wid = lax.axis_index("s") * 2 + lax.axis_index("c")
    r0 = wid * Q
    r_hi = r0 + Q
    neg_vec = jnp.full((16,), NEG_INF, jnp.float32)

    for j in range(NV):
        negbuf[pl.ds(j * 16, 16)] = neg_vec

    def fill_neg(lo, n):
        def fb(k, cc):
            for j in range(NV):
                staging[lo + k, pl.ds(j * 16, 16)] = neg_vec
            return cc
        lax.fori_loop(0, n, fb, 0)

    fill_neg(jnp.int32(0), jnp.int32(W))

    @pl.when(wid > 0)
    def _():
        pltpu.sync_copy(b_hbm.at[pl.ds(r0 - 16, 16)], prevbuf)

    @pl.when(wid == 0)
    def _():
        prevbuf[...] = jnp.full((16,), -1, jnp.int32)

    prev = prevbuf[...][15]
    s_off = prev + 1

    def write_empty(s2, carry):
        pltpu.sync_copy(negbuf, o_hbm.at[s2])
        return carry

    def flush(cur, vbase, acc):
        v = cur - s_off

        def need_drain(vb):
            return v >= vb + HW

        def do_drain(vb):
            h0 = pl.multiple_of(vb & (W - 1), HW)
            for k in range(HW // 16):
                sidx[pl.ds(16 * k, 16)] = (lax.iota(jnp.int32, 16)
                                           + (s_off + vb + 16 * k))
            pltpu.async_copy(staging.at[pl.ds(h0, HW)], o_hbm.at[sidx],
                             stsem).wait()
            fill_neg(h0, jnp.int32(HW))
            return vb + HW

        vbase = lax.while_loop(need_drain, do_drain, vbase)
        slot = v & (W - 1)
        for j in range(NV):
            staging[slot, pl.ds(j * 16, 16)] = acc[j]
        return vbase

    def start_dmas(chunk, xbuf, idbuf, sem):
        pltpu.async_copy(x_hbm.at[pl.ds(chunk * C, C)], xbuf, sem)
        pltpu.async_copy(b_hbm.at[pl.ds(chunk * C, C)],
                         idbuf.at[pl.ds(16, C)], sem)

    def wait_dmas(xbuf, idbuf, sem):
        pltpu.make_async_copy(x_hbm.at[pl.ds(0, C)], xbuf, sem).wait()
        pltpu.make_async_copy(b_hbm.at[pl.ds(0, C)],
                              idbuf.at[pl.ds(16, C)], sem).wait()

    def process_chunk(xbuf, idbuf, chunk, st):
        cur, mode, vbase, vfl, last_id, acc = st
        base = chunk * C
        rstart = jnp.maximum(r0 - base, 0)

        idbuf[pl.ds(0, 16)] = jnp.zeros((16,), jnp.int32) + last_id

        off = jnp.int32(0)
        for g in range(C // 16):
            idv = idbuf[pl.ds(16 + 16 * g, 16)]
            idp = idbuf[pl.ds(15 + 16 * g, 16)]
            riota = lax.iota(jnp.int32, 16) + (16 * g)
            m = (idv != idp) & (riota >= rstart)
            plsc.store_compressed(bpos.at[pl.ds(off, 16)], riota, mask=m)
            off = off + plsc.all_reduce_population_count(m)[0]
        nb = off

        def vmax_body(t, a):
            return tuple(
                jnp.maximum(a[j], xbuf[t, pl.ds(16 * j, 16)])
                for j in range(NV))

        def accumulate(lo, hi, a):
            return plsc.parallel_loop(lo, hi, carry=a, unroll=4)(vmax_body)

        def bloop(i, st2):
            pos, cur, mode, vbase, vfl, acc = st2
            b = bpos[pl.ds(i, 16)][0]
            acc = accumulate(pos, b, acc)
            sid = idbuf[pl.ds(16 + b, 16)][0]
            in_range = (base + b) < r_hi
            is_acc = mode == 1
            not_done = mode != 2

            vbase2 = lax.cond(
                is_acc,
                lambda: flush(cur, vbase, acc),
                lambda: vbase)
            vfl = jnp.where(is_acc, cur - s_off, vfl)

            started = not_done & in_range
            new_mode = jnp.where(not_done,
                                 jnp.where(in_range, jnp.int32(1),
                                           jnp.int32(2)),
                                 mode)
            new_cur = jnp.where(started, sid, cur)
            acc = tuple(jnp.where(started, neg_vec, a) for a in acc)
            return b, new_cur, new_mode, vbase2, vfl, acc

        pos, cur, mode, vbase, vfl, acc = lax.fori_loop(
            0, nb, bloop, (rstart, cur, mode, vbase, vfl, acc))
        acc = accumulate(pos, C, acc)
        last_id = idbuf[pl.ds(C, 16)][15]
        return cur, mode, vbase, vfl, last_id, acc

    def chunk_cond(carry):
        chunk = carry[0]
        mode = carry[2]
        return (mode != 2) & (chunk < NCHUNK)

    def chunk_body(carry):
        chunk, cur, mode, vbase, vfl, last_id = carry[:6]
        acc = carry[6:]
        st = (cur, mode, vbase, vfl, last_id, acc)
        wait_dmas(xbuf0, idbuf0, sem0)
        st = process_chunk(xbuf0, idbuf0, chunk, st)
        wait_dmas(xbuf1, idbuf1, sem1)
        st = process_chunk(xbuf1, idbuf1, chunk + 1, st)
        cur, mode, vbase, vfl, last_id, acc = st

        @pl.when((mode != 2) & (chunk + 2 < NCHUNK))
        def _():
            start_dmas(chunk + 2, xbuf0, idbuf0, sem0)
            start_dmas(chunk + 3, xbuf1, idbuf1, sem1)

        return (chunk + 2, cur, mode, vbase, vfl, last_id) + acc

    c0 = ((r0 // C) // 2) * 2
    start_dmas(c0, xbuf0, idbuf0, sem0)
    start_dmas(c0 + 1, xbuf1, idbuf1, sem1)
    acc0 = tuple(neg_vec for _ in range(NV))
    carry = (c0, jnp.int32(0), jnp.int32(0), jnp.int32(0), jnp.int32(-1),
             prev) + acc0
    carry = lax.while_loop(chunk_cond, chunk_body, carry)
    cur, mode, vbase, vfl = carry[1], carry[2], carry[3], carry[4]
    acc = carry[6:]

    @pl.when(mode == 1)
    def _():
        vb2 = flush(cur, vbase, acc)
        lax.fori_loop(cur + 1, S, write_empty, 0)

    v_end = jnp.where(mode == 1, cur - s_off, vfl)
    vbase_end = jnp.maximum(vbase,
                            jnp.where(v_end >= vbase + HW,
                                      ((v_end - HW) // HW + 1) * HW, vbase))

    @pl.when(v_end >= 0)
    def _():
        nrem = v_end - vbase_end + 1

        def dr(i, cc):
            vv = vbase_end + i
            pltpu.async_copy(staging.at[vv & (W - 1)],
                             o_hbm.at[s_off + vv], stsem)
            return cc

        lax.fori_loop(0, nrem, dr, 0)

        def drw(i, cc):
            pltpu.make_async_copy(staging.at[0], o_hbm.at[0], stsem).wait()
            return cc

        lax.fori_loop(0, nrem, drw, 0)


def kernel(x, batch):
    mesh = plsc.VectorSubcoreMesh(core_axis_name="c", subcore_axis_name="s")
    cp = pltpu.CompilerParams()
    if "needs_layout_passes" in pltpu.CompilerParams.__dataclass_fields__:
        cp = dataclasses.replace(cp, needs_layout_passes=False)
    f = pl.kernel(
        _body,
        compiler_params=cp,
        out_type=jax.ShapeDtypeStruct((S, D), jnp.float32),
        mesh=mesh,
        scratch_types=[
            pltpu.VMEM((C, D), jnp.float32),
            pltpu.VMEM((C, D), jnp.float32),
            pltpu.VMEM((C + 32,), jnp.int32),
            pltpu.VMEM((C + 32,), jnp.int32),
            pltpu.VMEM((C + 16,), jnp.int32),
            pltpu.VMEM((W, D), jnp.float32),
            pltpu.VMEM((HW,), jnp.int32),
            pltpu.VMEM((D,), jnp.float32),
            pltpu.VMEM((16,), jnp.int32),
            pltpu.SemaphoreType.DMA,
            pltpu.SemaphoreType.DMA,
            pltpu.SemaphoreType.DMA,
        ],
    )
    return f(x, batch)


# speedup vs baseline: 1.0177x; 1.0177x over previous
import dataclasses

import jax
import jax.numpy as jnp
from jax import lax
from jax.experimental import pallas as pl
from jax.experimental.pallas import tpu as pltpu
from jax.experimental.pallas import tpu_sc as plsc

N = 320000
D = 128
S = 10000
NW = 32
Q = N // NW
C = 400
NCHUNK = N // C
NV = D // 16
W = 128
HW = W // 2

NEG_INF = float("-inf")


def _body(x_hbm, b_hbm, o_hbm, xbuf0, xbuf1, idbuf0, idbuf1, bpos, staging,
          sidx, negbuf, prevbuf, sem0, sem1, stsem):
    wid = lax.axis_index("s") * 2 + lax.axis_index("c")
    r0 = wid * Q
    r_hi = r0 + Q
    neg_vec = jnp.full((16,), NEG_INF, jnp.float32)

    for j in range(NV):
        negbuf[pl.ds(j * 16, 16)] = neg_vec

    def fill_neg(lo, n):
        def fb(k, cc):
            for j in range(NV):
                staging[lo + k, pl.ds(j * 16, 16)] = neg_vec
            return cc
        lax.fori_loop(0, n, fb, 0)

    fill_neg(jnp.int32(0), jnp.int32(W))

    @pl.when(wid > 0)
    def _():
        pltpu.sync_copy(b_hbm.at[pl.ds(r0 - 16, 16)], prevbuf)

    @pl.when(wid == 0)
    def _():
        prevbuf[...] = jnp.full((16,), -1, jnp.int32)

    prev = prevbuf[...][15]
    s_off = prev + 1

    def write_empty(s2, carry):
        pltpu.sync_copy(negbuf, o_hbm.at[s2])
        return carry

    def drain_to(v, vbase):
        def need_drain(vb):
            return v >= vb + HW

        def do_drain(vb):
            h0 = pl.multiple_of(vb & (W - 1), HW)
            for k in range(HW // 16):
                sidx[pl.ds(16 * k, 16)] = (lax.iota(jnp.int32, 16)
                                           + (s_off + vb + 16 * k))
            pltpu.async_copy(staging.at[pl.ds(h0, HW)], o_hbm.at[sidx],
                             stsem).wait()
            fill_neg(h0, jnp.int32(HW))
            return vb + HW

        return lax.while_loop(need_drain, do_drain, vbase)

    def start_dmas(chunk, xbuf, idbuf, sem):
        pltpu.async_copy(x_hbm.at[pl.ds(chunk * C, C)], xbuf, sem)
        pltpu.async_copy(b_hbm.at[pl.ds(chunk * C, C)],
                         idbuf.at[pl.ds(16, C)], sem)

    def wait_dmas(xbuf, idbuf, sem):
        pltpu.make_async_copy(x_hbm.at[pl.ds(0, C)], xbuf, sem).wait()
        pltpu.make_async_copy(b_hbm.at[pl.ds(0, C)],
                              idbuf.at[pl.ds(16, C)], sem).wait()

    def process_chunk(xbuf, idbuf, chunk, st):
        cur, mode, vbase, vfl, last_id = st
        base = chunk * C
        rstart = jnp.maximum(r0 - base, 0)

        idbuf[pl.ds(0, 16)] = jnp.zeros((16,), jnp.int32) + last_id

        off = jnp.int32(0)
        for g in range(C // 16):
            idv = idbuf[pl.ds(16 + 16 * g, 16)]
            idp = idbuf[pl.ds(15 + 16 * g, 16)]
            riota = lax.iota(jnp.int32, 16) + (16 * g)
            m = (idv != idp) & (riota >= rstart)
            plsc.store_compressed(bpos.at[pl.ds(off, 16)], riota, mask=m)
            off = off + plsc.all_reduce_population_count(m)[0]
        nb = off

        def vmax_body(t, a):
            return tuple(
                jnp.maximum(a[j], xbuf[t, pl.ds(16 * j, 16)])
                for j in range(NV))

        def accumulate(lo, hi, a):
            return plsc.parallel_loop(lo, hi, carry=a, unroll=4)(vmax_body)

        def stage_interval(cur, lo, hi):
            slot = (cur - s_off) & (W - 1)
            a = tuple(staging[slot, pl.ds(16 * j, 16)] for j in range(NV))
            a = accumulate(lo, hi, a)
            for j in range(NV):
                staging[slot, pl.ds(16 * j, 16)] = a[j]

        def bloop(i, st2):
            pos, cur, mode, vbase, vfl = st2
            b = bpos[pl.ds(i, 16)][0]

            @pl.when(mode == 1)
            def _():
                stage_interval(cur, pos, b)

            sid = idbuf[pl.ds(16 + b, 16)][0]
            in_range = (base + b) < r_hi
            not_done = mode != 2
            started = not_done & in_range
            vbase = lax.cond(
                started,
                lambda: drain_to(sid - s_off, vbase),
                lambda: vbase)
            new_mode = jnp.where(not_done,
                                 jnp.where(in_range, jnp.int32(1),
                                           jnp.int32(2)),
                                 mode)
            new_cur = jnp.where(started, sid, cur)
            vfl = jnp.where(started, sid - s_off, vfl)
            return b, new_cur, new_mode, vbase, vfl

        pos, cur, mode, vbase, vfl = lax.fori_loop(
            0, nb, bloop, (rstart, cur, mode, vbase, vfl))

        @pl.when(mode == 1)
        def _():
            stage_interval(cur, pos, jnp.int32(C))

        last_id = idbuf[pl.ds(C, 16)][15]
        return cur, mode, vbase, vfl, last_id

    def chunk_cond(carry):
        chunk = carry[0]
        mode = carry[2]
        return (mode != 2) & (chunk < NCHUNK)

    def chunk_body(carry):
        chunk, cur, mode, vbase, vfl, last_id = carry
        st = (cur, mode, vbase, vfl, last_id)
        wait_dmas(xbuf0, idbuf0, sem0)
        st = process_chunk(xbuf0, idbuf0, chunk, st)
        wait_dmas(xbuf1, idbuf1, sem1)
        st = process_chunk(xbuf1, idbuf1, chunk + 1, st)
        cur, mode, vbase, vfl, last_id = st

        @pl.when((mode != 2) & (chunk + 2 < NCHUNK))
        def _():
            start_dmas(chunk + 2, xbuf0, idbuf0, sem0)
            start_dmas(chunk + 3, xbuf1, idbuf1, sem1)

        return (chunk + 2, cur, mode, vbase, vfl, last_id)

    c0 = ((r0 // C) // 2) * 2
    start_dmas(c0, xbuf0, idbuf0, sem0)
    start_dmas(c0 + 1, xbuf1, idbuf1, sem1)
    carry = (c0, jnp.int32(0), jnp.int32(0), jnp.int32(0), jnp.int32(-1),
             prev)
    carry = lax.while_loop(chunk_cond, chunk_body, carry)
    cur, mode, vbase, vfl = carry[1], carry[2], carry[3], carry[4]

    @pl.when(mode == 1)
    def _():
        lax.fori_loop(cur + 1, S, write_empty, 0)

    @pl.when(vfl >= 0)
    def _():
        nrem = vfl - vbase + 1

        def dr(i, cc):
            vv = vbase + i
            pltpu.async_copy(staging.at[vv & (W - 1)],
                             o_hbm.at[s_off + vv], stsem)
            return cc

        lax.fori_loop(0, nrem, dr, 0)

        def drw(i, cc):
            pltpu.make_async_copy(staging.at[0], o_hbm.at[0], stsem).wait()
            return cc

        lax.fori_loop(0, nrem, drw, 0)


def kernel(x, batch):
    mesh = plsc.VectorSubcoreMesh(core_axis_name="c", subcore_axis_name="s")
    cp = pltpu.CompilerParams()
    if "needs_layout_passes" in pltpu.CompilerParams.__dataclass_fields__:
        cp = dataclasses.replace(cp, needs_layout_passes=False)
    f = pl.kernel(
        _body,
        compiler_params=cp,
        out_type=jax.ShapeDtypeStruct((S, D), jnp.float32),
        mesh=mesh,
        scratch_types=[
            pltpu.VMEM((C, D), jnp.float32),
            pltpu.VMEM((C, D), jnp.float32),
            pltpu.VMEM((C + 32,), jnp.int32),
            pltpu.VMEM((C + 32,), jnp.int32),
            pltpu.VMEM((C + 16,), jnp.int32),
            pltpu.VMEM((W, D), jnp.float32),
            pltpu.VMEM((HW,), jnp.int32),
            pltpu.VMEM((D,), jnp.float32),
            pltpu.VMEM((16,), jnp.int32),
            pltpu.SemaphoreType.DMA,
            pltpu.SemaphoreType.DMA,
            pltpu.SemaphoreType.DMA,
        ],
    )
    return f(x, batch)


# speedup vs baseline: 1.0325x; 1.0145x over previous
import dataclasses

import jax
import jax.numpy as jnp
from jax import lax
from jax.experimental import pallas as pl
from jax.experimental.pallas import tpu as pltpu
from jax.experimental.pallas import tpu_sc as plsc

N = 320000
D = 128
S = 10000
NW = 32
Q = N // NW
C = 400
NCHUNK = N // C
NV = D // 16
W = 128
HW = W // 2

NEG_INF = float("-inf")


def _body(x_hbm, b_hbm, o_hbm, xbuf0, xbuf1, idbuf0, idbuf1, bpos, staging,
          sidx, negbuf, prevbuf, sem0, sem1, stsem):
    wid = lax.axis_index("s") * 2 + lax.axis_index("c")
    r0 = wid * Q
    r_hi = r0 + Q
    neg_vec = jnp.full((16,), NEG_INF, jnp.float32)

    for j in range(NV):
        negbuf[pl.ds(j * 16, 16)] = neg_vec

    def fill_neg(lo, n):
        def fb(k, cc):
            for j in range(NV):
                staging[lo + k, pl.ds(j * 16, 16)] = neg_vec
            return cc
        lax.fori_loop(0, n, fb, 0)

    fill_neg(jnp.int32(0), jnp.int32(W))

    @pl.when(wid > 0)
    def _():
        pltpu.sync_copy(b_hbm.at[pl.ds(r0 - 16, 16)], prevbuf)

    @pl.when(wid == 0)
    def _():
        prevbuf[...] = jnp.full((16,), -1, jnp.int32)

    prev = prevbuf[...][15]
    s_off = prev + 1

    def write_empty(s2, carry):
        pltpu.sync_copy(negbuf, o_hbm.at[s2])
        return carry

    def drain_to(v, vbase):
        def need_drain(vb):
            return v >= vb + HW

        def do_drain(vb):
            h0 = pl.multiple_of(vb & (W - 1), HW)
            for k in range(HW // 16):
                sidx[pl.ds(16 * k, 16)] = (lax.iota(jnp.int32, 16)
                                           + (s_off + vb + 16 * k))
            pltpu.async_copy(staging.at[pl.ds(h0, HW)], o_hbm.at[sidx],
                             stsem).wait()
            fill_neg(h0, jnp.int32(HW))
            return vb + HW

        return lax.while_loop(need_drain, do_drain, vbase)

    def start_dmas(chunk, xbuf, idbuf, sem):
        pltpu.async_copy(x_hbm.at[pl.ds(chunk * C, C)], xbuf, sem)
        pltpu.async_copy(b_hbm.at[pl.ds(chunk * C, C)],
                         idbuf.at[pl.ds(16, C)], sem)

    def wait_dmas(xbuf, idbuf, sem):
        pltpu.make_async_copy(x_hbm.at[pl.ds(0, C)], xbuf, sem).wait()
        pltpu.make_async_copy(b_hbm.at[pl.ds(0, C)],
                              idbuf.at[pl.ds(16, C)], sem).wait()

    def process_chunk(xbuf, idbuf, chunk, st):
        cur, mode, vbase, vfl, last_id = st
        base = chunk * C
        rstart = jnp.maximum(r0 - base, 0)

        idbuf[pl.ds(0, 16)] = jnp.zeros((16,), jnp.int32) + last_id


        off = jnp.int32(0)
        for g in range(C // 16):
            idv = idbuf[pl.ds(16 + 16 * g, 16)]
            idp = idbuf[pl.ds(15 + 16 * g, 16)]
            riota = lax.iota(jnp.int32, 16) + (16 * g)
            m = (idv != idp) & (riota >= rstart)
            plsc.store_compressed(bpos.at[pl.ds(off, 16)], riota, mask=m)
            off = off + plsc.all_reduce_population_count(m)[0]
        nb = off

        def vmax_body(t, a):
            return tuple(
                jnp.maximum(a[j], xbuf[t, pl.ds(16 * j, 16)])
                for j in range(NV))

        def accumulate(lo, hi, a):
            return plsc.parallel_loop(lo, hi, carry=a, unroll=4)(vmax_body)

        def stage_interval(cur, lo, hi):
            slot = (cur - s_off) & (W - 1)
            a = tuple(staging[slot, pl.ds(16 * j, 16)] for j in range(NV))
            a = accumulate(lo, hi, a)
            for j in range(NV):
                staging[slot, pl.ds(16 * j, 16)] = a[j]

        def bloop(i, st2):
            pos, cur, mode, vbase, vfl, b = st2
            b_next = bpos[pl.ds(i + 1, 16)][0]
            sid = idbuf[pl.ds(16 + b, 16)][0]

            @pl.when(mode == 1)
            def _():
                stage_interval(cur, pos, b)

            in_range = (base + b) < r_hi
            not_done = mode != 2
            started = not_done & in_range
            vbase = lax.cond(
                started,
                lambda: drain_to(sid - s_off, vbase),
                lambda: vbase)
            new_mode = jnp.where(not_done,
                                 jnp.where(in_range, jnp.int32(1),
                                           jnp.int32(2)),
                                 mode)
            new_cur = jnp.where(started, sid, cur)
            vfl = jnp.where(started, sid - s_off, vfl)
            return b, new_cur, new_mode, vbase, vfl, b_next

        b0 = bpos[pl.ds(0, 16)][0]
        pos, cur, mode, vbase, vfl, _b = lax.fori_loop(
            0, nb, bloop, (rstart, cur, mode, vbase, vfl, b0))

        @pl.when(mode == 1)
        def _():
            stage_interval(cur, pos, jnp.int32(C))

        last_id = idbuf[pl.ds(C, 16)][15]
        return cur, mode, vbase, vfl, last_id

    def chunk_cond(carry):
        chunk = carry[0]
        mode = carry[2]
        return (mode != 2) & (chunk < NCHUNK)

    def chunk_body(carry):
        chunk, cur, mode, vbase, vfl, last_id = carry
        st = (cur, mode, vbase, vfl, last_id)
        wait_dmas(xbuf0, idbuf0, sem0)
        st = process_chunk(xbuf0, idbuf0, chunk, st)
        wait_dmas(xbuf1, idbuf1, sem1)
        st = process_chunk(xbuf1, idbuf1, chunk + 1, st)
        cur, mode, vbase, vfl, last_id = st

        @pl.when((mode != 2) & (chunk + 2 < NCHUNK))
        def _():
            start_dmas(chunk + 2, xbuf0, idbuf0, sem0)
            start_dmas(chunk + 3, xbuf1, idbuf1, sem1)

        return (chunk + 2, cur, mode, vbase, vfl, last_id)

    c0 = ((r0 // C) // 2) * 2
    start_dmas(c0, xbuf0, idbuf0, sem0)
    start_dmas(c0 + 1, xbuf1, idbuf1, sem1)
    carry = (c0, jnp.int32(0), jnp.int32(0), jnp.int32(0), jnp.int32(-1),
             prev)
    carry = lax.while_loop(chunk_cond, chunk_body, carry)
    cur, mode, vbase, vfl = carry[1], carry[2], carry[3], carry[4]

    @pl.when(mode == 1)
    def _():
        lax.fori_loop(cur + 1, S, write_empty, 0)

    @pl.when(vfl >= 0)
    def _():
        nrem = vfl - vbase + 1

        def dr(i, cc):
            vv = vbase + i
            pltpu.async_copy(staging.at[vv & (W - 1)],
                             o_hbm.at[s_off + vv], stsem)
            return cc

        lax.fori_loop(0, nrem, dr, 0)

        def drw(i, cc):
            pltpu.make_async_copy(staging.at[0], o_hbm.at[0], stsem).wait()
            return cc

        lax.fori_loop(0, nrem, drw, 0)


def kernel(x, batch):
    mesh = plsc.VectorSubcoreMesh(core_axis_name="c", subcore_axis_name="s")
    cp = pltpu.CompilerParams()
    if "needs_layout_passes" in pltpu.CompilerParams.__dataclass_fields__:
        cp = dataclasses.replace(cp, needs_layout_passes=False)
    f = pl.kernel(
        _body,
        compiler_params=cp,
        out_type=jax.ShapeDtypeStruct((S, D), jnp.float32),
        mesh=mesh,
        scratch_types=[
            pltpu.VMEM((C, D), jnp.float32),
            pltpu.VMEM((C, D), jnp.float32),
            pltpu.VMEM((C + 32,), jnp.int32),
            pltpu.VMEM((C + 32,), jnp.int32),
            pltpu.VMEM((C + 16,), jnp.int32),
            pltpu.VMEM((W, D), jnp.float32),
            pltpu.VMEM((HW,), jnp.int32),
            pltpu.VMEM((D,), jnp.float32),
            pltpu.VMEM((16,), jnp.int32),
            pltpu.SemaphoreType.DMA,
            pltpu.SemaphoreType.DMA,
            pltpu.SemaphoreType.DMA,
        ],
    )
    return f(x, batch)
